# Initial kernel scaffold; baseline (speedup 1.0000x reference)
import functools

import jax
import jax.numpy as jnp
from jax import lax
from jax.experimental import pallas as pl
from jax.experimental.pallas import tpu as pltpu
from jax.experimental.pallas import tpu_sc as plsc

N = 10000
E = 320000
D = 128

NC = 2
NS = 16
NW = NC * NS

K = 125
CHUNKS = E // K
CPW = CHUNKS // NW
EPW = E // NW
RPT = N // NS

_MESH = plsc.VectorSubcoreMesh(core_axis_name="c", subcore_axis_name="s")


def _degree_body(ei_hbm, out_hbm, sbuf, dbuf, ho, hi):
    cid = lax.axis_index("c")
    sid = lax.axis_index("s")
    w = sid * NC + cid
    base = w * EPW
    pltpu.sync_copy(ei_hbm.at[0, pl.ds(base, EPW)], sbuf)
    pltpu.sync_copy(ei_hbm.at[1, pl.ds(base, EPW)], dbuf)

    zeros = jnp.zeros((16,), jnp.float32)

    @pl.loop(0, N // 16)
    def _(i):
        ho[pl.ds(i * 16, 16)] = zeros
        hi[pl.ds(i * 16, 16)] = zeros

    ones = jnp.ones((16,), jnp.float32)

    @pl.loop(0, EPW // 16)
    def _(i):
        plsc.addupdate_scatter(ho, [sbuf[pl.ds(i * 16, 16)]], ones)
        plsc.addupdate_scatter(hi, [dbuf[pl.ds(i * 16, 16)]], ones)

    pltpu.sync_copy(ho, out_hbm.at[0, w])
    pltpu.sync_copy(hi, out_hbm.at[1, w])


_degree_kernel = functools.partial(
    pl.kernel,
    out_type=jax.ShapeDtypeStruct((2, NW, N), jnp.float32),
    mesh=_MESH,
    scratch_types=[
        pltpu.VMEM((EPW,), jnp.int32),
        pltpu.VMEM((EPW,), jnp.int32),
        pltpu.VMEM((N,), jnp.float32),
        pltpu.VMEM((N,), jnp.float32),
    ],
)(_degree_body)


def _prep_body(hist_ref, feat_ref, fs_ref, nd_ref):
    deg_o = jnp.sum(hist_ref[0], axis=1, keepdims=True)
    deg_i = jnp.sum(hist_ref[1], axis=1, keepdims=True)
    norm_src = lax.rsqrt(jnp.maximum(deg_o, 1.0))
    norm_dst = lax.rsqrt(jnp.maximum(deg_i, 1.0))
    fs_ref[...] = feat_ref[...] * norm_src
    nd_ref[...] = norm_dst


_prep_kernel = pl.pallas_call(
    _prep_body,
    out_shape=[
        jax.ShapeDtypeStruct((N, D), jnp.float32),
        jax.ShapeDtypeStruct((N, 1), jnp.float32),
    ],
)


def _agg_body(fs_hbm, ei3_hbm, ef_hbm, z_hbm, out_hbm,
              sidx, didx, efb, fsb, hacc):
    cid = lax.axis_index("c")
    sid = lax.axis_index("s")
    w = sid * NC + cid
    pltpu.sync_copy(ei3_hbm.at[0, pl.ds(w * CPW, CPW)], sidx)
    pltpu.sync_copy(ei3_hbm.at[1, pl.ds(w * CPW, CPW)], didx)
    pltpu.sync_copy(z_hbm, hacc.at[pl.ds(sid * RPT, RPT)])
    plsc.subcore_barrier()

    @pl.loop(0, CPW)
    def _(g):
        gg = w * CPW + g
        pltpu.sync_copy(ef_hbm.at[pl.ds(gg * K, K)], efb)
        pltpu.sync_copy(fs_hbm.at[sidx.at[g]], fsb)
        pltpu.sync_copy(efb, hacc.at[didx.at[g]], add=True)
        pltpu.sync_copy(fsb, hacc.at[didx.at[g]], add=True)

    plsc.subcore_barrier()
    pltpu.sync_copy(hacc.at[pl.ds(sid * RPT, RPT)],
                    out_hbm.at[cid, pl.ds(sid * RPT, RPT)])


_agg_kernel = functools.partial(
    pl.kernel,
    out_type=jax.ShapeDtypeStruct((NC, N, D), jnp.float32),
    mesh=_MESH,
    scratch_types=[
        pltpu.VMEM((CPW, K), jnp.int32),
        pltpu.VMEM((CPW, K), jnp.int32),
        pltpu.VMEM((K, D), jnp.float32),
        pltpu.VMEM((K, D), jnp.float32),
        pltpu.VMEM_SHARED((N, D), jnp.float32),
    ],
)(_agg_body)


def _final_body(h_ref, w_ref, b_ref, nd_ref, o_ref):
    h = h_ref[0] + h_ref[1]
    r = jnp.dot(h, w_ref[...], preferred_element_type=jnp.float32)
    o_ref[...] = r * nd_ref[...] + b_ref[...]


_final_kernel = pl.pallas_call(
    _final_body,
    out_shape=jax.ShapeDtypeStruct((N, D), jnp.float32),
)


def kernel(feat, edge_index, edgeFeat, weight, bias):
    hist = _degree_kernel(edge_index)
    hist_t = hist.transpose(0, 2, 1)
    feat_src, norm_dst = _prep_kernel(hist_t, feat)
    ei3 = edge_index.reshape(2, CHUNKS, K)
    zrows = jnp.zeros((RPT, D), jnp.float32)
    h_part = _agg_kernel(feat_src, ei3, edgeFeat, zrows)
    return _final_kernel(h_part, weight, bias.reshape(1, D), norm_dst)


# speedup vs baseline: 4.5320x; 4.5320x over previous
import functools

import jax
import jax.numpy as jnp
from jax import lax
from jax.experimental import pallas as pl
from jax.experimental.pallas import tpu as pltpu
from jax.experimental.pallas import tpu_sc as plsc

N = 10000
E = 320000
D = 128

NC = 2
NS = 16
NW = NC * NS

K = 100
CHUNKS = E // K
CPW = CHUNKS // NW
EPW = E // NW
RPT = N // NS

_MESH = plsc.VectorSubcoreMesh(core_axis_name="c", subcore_axis_name="s")


def _degree_body(src_hbm, dst_hbm, out_hbm, sbuf, dbuf, ho, hi):
    cid = lax.axis_index("c")
    sid = lax.axis_index("s")
    w = sid * NC + cid
    base = w * EPW
    pltpu.sync_copy(src_hbm.at[pl.ds(base, EPW)], sbuf)
    pltpu.sync_copy(dst_hbm.at[pl.ds(base, EPW)], dbuf)

    zeros = jnp.zeros((16,), jnp.float32)

    @pl.loop(0, N // 16)
    def _(i):
        ho[pl.ds(i * 16, 16)] = zeros
        hi[pl.ds(i * 16, 16)] = zeros

    ones = jnp.ones((16,), jnp.float32)

    @pl.loop(0, EPW // 16)
    def _(i):
        plsc.addupdate_scatter(ho, [sbuf[pl.ds(i * 16, 16)]], ones)
        plsc.addupdate_scatter(hi, [dbuf[pl.ds(i * 16, 16)]], ones)

    pltpu.sync_copy(ho, out_hbm.at[0, w])
    pltpu.sync_copy(hi, out_hbm.at[1, w])


_degree_kernel = functools.partial(
    pl.kernel,
    out_type=jax.ShapeDtypeStruct((2, NW, N), jnp.float32),
    mesh=_MESH,
    compiler_params=pltpu.CompilerParams(needs_layout_passes=False),
    scratch_types=[
        pltpu.VMEM((EPW,), jnp.int32),
        pltpu.VMEM((EPW,), jnp.int32),
        pltpu.VMEM((N,), jnp.float32),
        pltpu.VMEM((N,), jnp.float32),
    ],
)(_degree_body)


def _prep_body(hist_ref, feat_ref, fs_ref, nd_ref):
    deg_o = jnp.sum(hist_ref[0], axis=1, keepdims=True)
    deg_i = jnp.sum(hist_ref[1], axis=1, keepdims=True)
    norm_src = lax.rsqrt(jnp.maximum(deg_o, 1.0))
    norm_dst = lax.rsqrt(jnp.maximum(deg_i, 1.0))
    fs_ref[...] = feat_ref[...] * norm_src
    nd_ref[...] = norm_dst


_prep_kernel = pl.pallas_call(
    _prep_body,
    out_shape=[
        jax.ShapeDtypeStruct((N, D), jnp.float32),
        jax.ShapeDtypeStruct((N, 1), jnp.float32),
    ],
)


def _agg_body(fs_hbm, s3_hbm, d3_hbm, ef_hbm, z_hbm, out_hbm,
              sidx, didx, efb, fsb, hacc):
    cid = lax.axis_index("c")
    sid = lax.axis_index("s")
    w = sid * NC + cid
    pltpu.sync_copy(s3_hbm.at[pl.ds(w * CPW, CPW)], sidx)
    pltpu.sync_copy(d3_hbm.at[pl.ds(w * CPW, CPW)], didx)
    pltpu.sync_copy(z_hbm, hacc.at[pl.ds(sid * RPT, RPT)])
    plsc.subcore_barrier()

    @pl.loop(0, CPW)
    def _(g):
        gg = w * CPW + g
        pltpu.sync_copy(ef_hbm.at[pl.ds(gg * K, K)], efb)
        pltpu.sync_copy(fs_hbm.at[sidx.at[g]], fsb)
        pltpu.sync_copy(efb, hacc.at[didx.at[g]], add=True)
        pltpu.sync_copy(fsb, hacc.at[didx.at[g]], add=True)

    plsc.subcore_barrier()
    pltpu.sync_copy(hacc.at[pl.ds(sid * RPT, RPT)],
                    out_hbm.at[cid, pl.ds(sid * RPT, RPT)])


_agg_kernel = functools.partial(
    pl.kernel,
    out_type=jax.ShapeDtypeStruct((NC, N, D), jnp.float32),
    mesh=_MESH,
    compiler_params=pltpu.CompilerParams(use_tc_tiling_on_sc=False),
    scratch_types=[
        pltpu.VMEM((CPW, K), jnp.int32),
        pltpu.VMEM((CPW, K), jnp.int32),
        pltpu.VMEM((K, D), jnp.float32),
        pltpu.VMEM((K, D), jnp.float32),
        pltpu.VMEM_SHARED((N, D), jnp.float32),
    ],
)(_agg_body)


def _final_body(h_ref, w_ref, b_ref, nd_ref, o_ref):
    h = h_ref[0] + h_ref[1]
    r = jnp.dot(h, w_ref[...], preferred_element_type=jnp.float32)
    o_ref[...] = r * nd_ref[...] + b_ref[...]


_final_kernel = pl.pallas_call(
    _final_body,
    out_shape=jax.ShapeDtypeStruct((N, D), jnp.float32),
)


def kernel(feat, edge_index, edgeFeat, weight, bias):
    src = edge_index[0]
    dst = edge_index[1]
    hist = _degree_kernel(src, dst)
    hist_t = hist.transpose(0, 2, 1)
    feat_src, norm_dst = _prep_kernel(hist_t, feat)
    zrows = jnp.zeros((RPT, D), jnp.float32)
    h_part = _agg_kernel(feat_src, src.reshape(CHUNKS, K),
                         dst.reshape(CHUNKS, K), edgeFeat, zrows)
    return _final_kernel(h_part, weight, bias.reshape(1, D), norm_dst)


# speedup vs baseline: 5.1694x; 1.1406x over previous
import functools

import jax
import jax.numpy as jnp
from jax import lax
from jax.experimental import pallas as pl
from jax.experimental.pallas import tpu as pltpu
from jax.experimental.pallas import tpu_sc as plsc

N = 10000
E = 320000
D = 128
DH = D // 2

NC = 2
NS = 16
NW = NC * NS

K = 125
KP = 128
CHUNKS = E // K
CPT = CHUNKS // NS
EPW = E // NW
RPT = N // NS

_MESH = plsc.VectorSubcoreMesh(core_axis_name="c", subcore_axis_name="s")


def _degree_body(src_hbm, dst_hbm, out_hbm, sbuf, dbuf, ho, hi):
    cid = lax.axis_index("c")
    sid = lax.axis_index("s")
    w = sid * NC + cid
    base = w * EPW
    pltpu.sync_copy(src_hbm.at[pl.ds(base, EPW)], sbuf)
    pltpu.sync_copy(dst_hbm.at[pl.ds(base, EPW)], dbuf)

    zeros = jnp.zeros((16,), jnp.float32)

    @pl.loop(0, N // 16)
    def _(i):
        ho[pl.ds(i * 16, 16)] = zeros
        hi[pl.ds(i * 16, 16)] = zeros

    ones = jnp.ones((16,), jnp.float32)

    @pl.loop(0, EPW // 16)
    def _(i):
        plsc.addupdate_scatter(ho, [sbuf[pl.ds(i * 16, 16)]], ones)
        plsc.addupdate_scatter(hi, [dbuf[pl.ds(i * 16, 16)]], ones)

    pltpu.sync_copy(ho, out_hbm.at[0, w])
    pltpu.sync_copy(hi, out_hbm.at[1, w])


_degree_kernel = functools.partial(
    pl.kernel,
    out_type=jax.ShapeDtypeStruct((2, NW, N), jnp.float32),
    mesh=_MESH,
    compiler_params=pltpu.CompilerParams(needs_layout_passes=False),
    scratch_types=[
        pltpu.VMEM((EPW,), jnp.int32),
        pltpu.VMEM((EPW,), jnp.int32),
        pltpu.VMEM((N,), jnp.float32),
        pltpu.VMEM((N,), jnp.float32),
    ],
)(_degree_body)


def _prep_body(hist_ref, feat_ref, fs2_ref, nd_ref):
    deg_o = jnp.sum(hist_ref[0], axis=1, keepdims=True)
    deg_i = jnp.sum(hist_ref[1], axis=1, keepdims=True)
    norm_src = lax.rsqrt(jnp.maximum(deg_o, 1.0))
    norm_dst = lax.rsqrt(jnp.maximum(deg_i, 1.0))
    scaled = feat_ref[...] * norm_src
    fs2_ref[pl.ds(0, N), :] = scaled[:, :DH]
    fs2_ref[pl.ds(N, N), :] = scaled[:, DH:]
    nd_ref[...] = norm_dst


_prep_kernel = pl.pallas_call(
    _prep_body,
    out_shape=[
        jax.ShapeDtypeStruct((2 * N, DH), jnp.float32),
        jax.ShapeDtypeStruct((N, 1), jnp.float32),
    ],
)


def _agg_body(fs2_hbm, s3_hbm, d3_hbm, ef_hbm, z_hbm, out_hbm,
              sidx,
              ef0, ef1, ef2, ef3,
              fs0, fs1, fs2, fs3,
              dx0, dx1, dx2, dx3,
              se0, se1, se2, se3,
              sg0, sg1, sg2, sg3,
              sd0, sd1, sd2, sd3,
              ss0, ss1, ss2, ss3,
              hacc):
    cid = lax.axis_index("c")
    sid = lax.axis_index("s")
    efb = (ef0, ef1, ef2, ef3)
    fsb = (fs0, fs1, fs2, fs3)
    dxb = (dx0, dx1, dx2, dx3)
    sef = (se0, se1, se2, se3)
    sfs = (sg0, sg1, sg2, sg3)
    sdx = (sd0, sd1, sd2, sd3)
    ssc = (ss0, ss1, ss2, ss3)

    g0 = sid * CPT

    pltpu.sync_copy(s3_hbm.at[pl.ds(g0, CPT)], sidx)
    off = jnp.full((16,), cid * N, jnp.int32)

    @pl.loop(0, CPT)
    def _(r):
        for k in range(KP // 16):
            sl = pl.ds(k * 16, 16)
            sidx[r, sl] = sidx[r, sl] + off

    pltpu.sync_copy(z_hbm, hacc.at[pl.ds(sid * RPT, RPT)])
    plsc.subcore_barrier()

    col = pl.ds(cid * DH, DH)

    def issue_loads(g, q):
        pltpu.async_copy(ef_hbm.at[pl.ds(g * K, K), col], efb[q], sef[q])
        pltpu.async_copy(fs2_hbm.at[sidx.at[g - g0]], fsb[q], sfs[q])
        pltpu.async_copy(d3_hbm.at[g], dxb[q], sdx[q])

    def wait_loads(g, q):
        pltpu.make_async_copy(ef_hbm.at[pl.ds(g * K, K), col],
                              efb[q], sef[q]).wait()
        pltpu.make_async_copy(fs2_hbm.at[sidx.at[g - g0]],
                              fsb[q], sfs[q]).wait()
        pltpu.make_async_copy(d3_hbm.at[g], dxb[q], sdx[q]).wait()

    def combine(q):
        eq = efb[q]
        fq = fsb[q]

        @pl.loop(0, K, unroll=5)
        def _(r):
            for k in range(DH // 16):
                sl = pl.ds(k * 16, 16)
                plsc.addupdate(fq.at[r, sl], eq[r, sl])

    def issue_scatter(q):
        pltpu.async_copy(fsb[q].at[pl.ds(0, K)], hacc.at[dxb[q]], ssc[q],
                         add=True)

    def wait_scatter(q):
        pltpu.make_async_copy(fsb[q].at[pl.ds(0, K)], hacc.at[dxb[q]],
                              ssc[q]).wait()

    issue_loads(g0 + 0, 0)
    issue_loads(g0 + 1, 1)

    issue_loads(g0 + 2, 2)
    wait_loads(g0 + 0, 0)
    combine(0)
    issue_scatter(0)
    issue_loads(g0 + 3, 3)
    wait_loads(g0 + 1, 1)
    combine(1)
    issue_scatter(1)

    @pl.loop(0, (CPT - 4) // 4)
    def _(t):
        gg = g0 + 2 + 4 * t
        for j, q in enumerate((2, 3, 0, 1)):
            g = gg + j
            qn = (q + 2) % 4
            wait_scatter(qn)
            issue_loads(g + 2, qn)
            wait_loads(g, q)
            combine(q)
            issue_scatter(q)

    wait_loads(g0 + CPT - 2, 2)
    combine(2)
    issue_scatter(2)
    wait_loads(g0 + CPT - 1, 3)
    combine(3)
    issue_scatter(3)
    for q in range(4):
        wait_scatter(q)

    plsc.subcore_barrier()
    pltpu.sync_copy(hacc.at[pl.ds(sid * RPT, RPT)],
                    out_hbm.at[pl.ds(sid * RPT, RPT), col])


_agg_kernel = functools.partial(
    pl.kernel,
    out_type=jax.ShapeDtypeStruct((N, D), jnp.float32),
    mesh=_MESH,
    compiler_params=pltpu.CompilerParams(use_tc_tiling_on_sc=False),
    scratch_types=(
        [pltpu.VMEM((CPT, KP), jnp.int32)]
        + [pltpu.VMEM((K, DH), jnp.float32) for _ in range(4)]
        + [pltpu.VMEM((KP, DH), jnp.float32) for _ in range(4)]
        + [pltpu.VMEM((K,), jnp.int32) for _ in range(4)]
        + [pltpu.SemaphoreType.DMA for _ in range(16)]
        + [pltpu.VMEM_SHARED((N, DH), jnp.float32)]
    ),
)(_agg_body)


def _final_body(h_ref, w_ref, b_ref, nd_ref, o_ref):
    r = jnp.dot(h_ref[...], w_ref[...], preferred_element_type=jnp.float32)
    o_ref[...] = r * nd_ref[...] + b_ref[...]


_final_kernel = pl.pallas_call(
    _final_body,
    out_shape=jax.ShapeDtypeStruct((N, D), jnp.float32),
)


def kernel(feat, edge_index, edgeFeat, weight, bias):
    src = edge_index[0]
    dst = edge_index[1]
    hist = _degree_kernel(src, dst)
    hist_t = hist.transpose(0, 2, 1)
    fs2, norm_dst = _prep_kernel(hist_t, feat)
    src3p = jnp.pad(src.reshape(CHUNKS, K), ((0, 0), (0, KP - K)))
    dst3 = dst.reshape(CHUNKS, K)
    zrows = jnp.zeros((RPT, DH), jnp.float32)
    h = _agg_kernel(fs2, src3p, dst3, edgeFeat, zrows)
    return _final_kernel(h, weight, bias.reshape(1, D), norm_dst)
